# baseline (speedup 1.0000x reference)
import jax
import jax.numpy as jnp
from jax.experimental import pallas as pl
from jax.experimental.pallas import tpu as pltpu

_B, _ED, _D, _H, _W = 1, 64, 64, 64, 64
_K = 1000
_KP = 1024
_N = _D * _H * _W
_NT = 8192
_HB = _NT // _W
_GRID = _N // _NT


def _nn_kernel(e_ref, ksq_ref, q_ref, idx_ref, dist_ref):
    e2 = e_ref[...]
    q3 = q_ref[...]
    q = jnp.concatenate([q3[:, j, :] for j in range(_HB)], axis=1)
    s = jax.lax.dot_general(e2, q,
                            (((1,), (0,)), ((), ())),
                            preferred_element_type=jnp.float32)
    s = s + ksq_ref[...]
    idx = jnp.argmin(s, axis=0).astype(jnp.int32)
    smin = jnp.min(s, axis=0)
    q_sq = jnp.sum(q * q, axis=0)
    d = jnp.sqrt(jnp.maximum(smin + q_sq, jnp.float32(0.0)))
    idx_ref[0, 0, :] = idx
    dist_ref[0, 0, :] = d


def kernel(embedded_data, embedding_matrix):
    q = embedded_data.reshape(_ED, _N // _W, _W)
    e2 = jnp.pad(embedding_matrix, ((0, _KP - _K), (0, 0))) * jnp.float32(-2.0)
    ksq = jnp.pad(jnp.sum(embedding_matrix * embedding_matrix, axis=1),
                  (0, _KP - _K), constant_values=3e38).reshape(_KP, 1)
    tokens, distances = pl.pallas_call(
        _nn_kernel,
        grid=(_GRID,),
        in_specs=[
            pl.BlockSpec((_KP, _ED), lambda i: (0, 0)),
            pl.BlockSpec((_KP, 1), lambda i: (0, 0)),
            pl.BlockSpec((_ED, _HB, _W), lambda i: (0, i, 0)),
        ],
        out_specs=[
            pl.BlockSpec((1, 1, _NT), lambda i: (i, 0, 0)),
            pl.BlockSpec((1, 1, _NT), lambda i: (i, 0, 0)),
        ],
        out_shape=[
            jax.ShapeDtypeStruct((_GRID, 1, _NT), jnp.int32),
            jax.ShapeDtypeStruct((_GRID, 1, _NT), jnp.float32),
        ],
        compiler_params=pltpu.CompilerParams(
            dimension_semantics=("parallel",)),
    )(e2, ksq, q)
    return tokens.reshape(_D, _H, _W), distances.reshape(_D, _H, _W)
